# baseline (speedup 1.0000x reference)
import functools

import jax
import jax.numpy as jnp
from jax import lax
from jax.experimental import pallas as pl
from jax.experimental.pallas import tpu as pltpu, tpu_sc as plsc

_ALPHA = 0.7
_BETA = 0.3
_SMOOTH = 1.0

_INFO = plsc.get_sparse_core_info()
_NC, _NS, _L = _INFO.num_cores, _INFO.num_subcores, _INFO.num_lanes
_NW = _NC * _NS
_NBUF = 4
_DSC = 16
_DBLK = 16


def _make_sc_sums(B, C, V, v_sc):
    per_w = v_sc // _NW
    ch = min(8192, per_w)
    n_ch = per_w // ch
    n_steps = B * n_ch
    mesh = plsc.VectorSubcoreMesh(core_axis_name="c", subcore_axis_name="s")
    scratch = (
        [pltpu.VMEM((ch,), jnp.float32) for _ in range(2)]
        + [pltpu.VMEM((ch,), jnp.int32) for _ in range(2)]
        + [pltpu.VMEM((B * 3 * _L,), jnp.float32)]
        + [pltpu.SemaphoreType.DMA for _ in range(4)]
    )

    @functools.partial(
        pl.kernel,
        mesh=mesh,
        out_type=jax.ShapeDtypeStruct((_NW * B * 3 * _L,), jnp.float32),
        scratch_types=scratch,
    )
    def sc_sums(x_hbm, t_hbm, out_hbm, xv0, xv1, tv0, tv1, outv,
                sx0, sx1, st0, st1):
        wid = lax.axis_index("s") * _NC + lax.axis_index("c")
        base = wid * per_w
        zero = jnp.zeros((_L,), jnp.float32)
        for r in range(B * 3):
            outv[pl.ds(r * _L, _L)] = zero

        def issue(i, xv, tv, sx, st):
            b = i // n_ch
            off = base + (i - b * n_ch) * ch
            pltpu.async_copy(
                x_hbm.at[pl.ds((b * C + (C - 1)) * V + off, ch)], xv, sx)
            pltpu.async_copy(t_hbm.at[pl.ds(b * V + off, ch)], tv, st)

        def process(i, xv, tv, sx, st):
            pltpu.make_async_copy(x_hbm.at[pl.ds(0, ch)], xv, sx).wait()
            pltpu.make_async_copy(t_hbm.at[pl.ds(0, ch)], tv, st).wait()

            def inner(j, carry):
                s, t, n = carry
                xs = xv[pl.ds(j * _L, _L)]
                ts = tv[pl.ds(j * _L, _L)]
                m = ts == (C - 1)
                s = s + xs
                t = t + jnp.where(m, xs, 0.0)
                n = n + jnp.where(m, 1.0, 0.0)
                return s, t, n

            S, T, N = lax.fori_loop(0, ch // _L, inner, (zero, zero, zero),
                                    unroll=4)
            boff = (i // n_ch) * 3 * _L
            outv[pl.ds(boff, _L)] = outv[pl.ds(boff, _L)] + S
            outv[pl.ds(boff + _L, _L)] = outv[pl.ds(boff + _L, _L)] + T
            outv[pl.ds(boff + 2 * _L, _L)] = outv[pl.ds(boff + 2 * _L, _L)] + N

        issue(0, xv0, tv0, sx0, st0)
        if n_steps > 1:
            issue(1, xv1, tv1, sx1, st1)

        def body(jj, _):
            i0 = 2 * jj
            for (i, xv, tv, sx, st) in ((i0, xv0, tv0, sx0, st0),
                                        (i0 + 1, xv1, tv1, sx1, st1)):
                process(i, xv, tv, sx, st)

                @pl.when(i + 2 < n_steps)
                def _(i=i, xv=xv, tv=tv, sx=sx, st=st):
                    issue(i + 2, xv, tv, sx, st)
            return 0

        lax.fori_loop(0, n_steps // 2, body, 0)
        pltpu.sync_copy(outv, out_hbm.at[pl.ds(wid * B * 3 * _L, B * 3 * _L)])

    return sc_sums


def _tc_sums_body(x_ref, t_ref, o_ref):
    d = pl.program_id(1)
    xb = x_ref[0, 0]
    m = (t_ref[0] == 3).astype(jnp.float32)
    xr = xb.reshape(-1, 8, 128)
    mr = m.reshape(-1, 8, 128)
    part = jnp.stack([xr.sum(0), (xr * mr).sum(0), mr.sum(0)])[None]

    @pl.when(d == 0)
    def _():
        o_ref[...] = jnp.zeros_like(o_ref)

    o_ref[...] += part


def kernel(inputs, targets):
    B, C, D, H, W = inputs.shape
    V = D * H * W
    v_sc = _DSC * H * W
    xf = inputs.reshape(-1)
    tf = targets.reshape(-1)
    sc_part = _make_sc_sums(B, C, V, v_sc)(xf, tf)
    inputs, targets, sc_part = lax.optimization_barrier(
        (inputs, targets, sc_part))
    d_off = _DSC // _DBLK
    tc_part = pl.pallas_call(
        _tc_sums_body,
        grid=(B, (D - _DSC) // _DBLK),
        in_specs=[
            pl.BlockSpec((1, 1, _DBLK, H, W),
                         lambda b, d: (b, C - 1, d + d_off, 0, 0)),
            pl.BlockSpec((1, _DBLK, H, W), lambda b, d: (b, d + d_off, 0, 0)),
        ],
        out_specs=pl.BlockSpec((1, 3, 8, 128), lambda b, d: (b, 0, 0, 0)),
        out_shape=jax.ShapeDtypeStruct((B, 3, 8, 128), jnp.float32),
    )(inputs, targets)
    sums = (sc_part.reshape(_NW, B, 3, _L).sum(axis=(0, 3))
            + tc_part.sum(axis=(2, 3)))
    S, T, N = sums[:, 0], sums[:, 1], sums[:, 2]
    tversky = (T + _SMOOTH) / (T + _ALPHA * (N - T) + _BETA * (S - T) + _SMOOTH)
    return -tversky.mean()


# speedup vs baseline: 1.2164x; 1.2164x over previous
import functools

import jax
import jax.numpy as jnp
from jax import lax
from jax.experimental import pallas as pl
from jax.experimental.pallas import tpu as pltpu, tpu_sc as plsc

_ALPHA = 0.7
_BETA = 0.3
_SMOOTH = 1.0

_INFO = plsc.get_sparse_core_info()
_NC, _NS, _L = _INFO.num_cores, _INFO.num_subcores, _INFO.num_lanes
_NW = _NC * _NS
_DSC = 16
_DBLK = 16


def _make_sc_sums(B, C, V, v_sc):
    per_w = v_sc // _NW
    ch = min(8192, per_w)
    n_ch = per_w // ch
    n_steps = B * n_ch
    mesh = plsc.VectorSubcoreMesh(core_axis_name="c", subcore_axis_name="s")
    scratch = (
        [pltpu.VMEM((ch,), jnp.float32) for _ in range(2)]
        + [pltpu.VMEM((ch,), jnp.int32) for _ in range(2)]
        + [pltpu.VMEM((B * 3 * _L,), jnp.float32)]
        + [pltpu.SemaphoreType.DMA for _ in range(4)]
    )

    @functools.partial(
        pl.kernel,
        mesh=mesh,
        out_type=jax.ShapeDtypeStruct((B * 3, _NW * _L), jnp.float32),
        scratch_types=scratch,
    )
    def sc_sums(x_hbm, t_hbm, out_hbm, xv0, xv1, tv0, tv1, outv,
                sx0, sx1, st0, st1):
        wid = lax.axis_index("s") * _NC + lax.axis_index("c")
        base = wid * per_w
        zero = jnp.zeros((_L,), jnp.float32)
        for r in range(B * 3):
            outv[pl.ds(r * _L, _L)] = zero

        def issue(i, xv, tv, sx, st):
            b = i // n_ch
            off = base + (i - b * n_ch) * ch
            pltpu.async_copy(
                x_hbm.at[pl.ds((b * C + (C - 1)) * V + off, ch)], xv, sx)
            pltpu.async_copy(t_hbm.at[pl.ds(b * V + off, ch)], tv, st)

        def process(i, xv, tv, sx, st):
            pltpu.make_async_copy(x_hbm.at[pl.ds(0, ch)], xv, sx).wait()
            pltpu.make_async_copy(t_hbm.at[pl.ds(0, ch)], tv, st).wait()

            def inner(j, carry):
                s, t, n = carry
                xs = xv[pl.ds(j * _L, _L)]
                ts = tv[pl.ds(j * _L, _L)]
                m = ts == (C - 1)
                s = s + xs
                t = t + jnp.where(m, xs, 0.0)
                n = n + jnp.where(m, 1.0, 0.0)
                return s, t, n

            S, T, N = lax.fori_loop(0, ch // _L, inner, (zero, zero, zero),
                                    unroll=4)
            boff = (i // n_ch) * 3 * _L
            outv[pl.ds(boff, _L)] = outv[pl.ds(boff, _L)] + S
            outv[pl.ds(boff + _L, _L)] = outv[pl.ds(boff + _L, _L)] + T
            outv[pl.ds(boff + 2 * _L, _L)] = outv[pl.ds(boff + 2 * _L, _L)] + N

        issue(0, xv0, tv0, sx0, st0)
        if n_steps > 1:
            issue(1, xv1, tv1, sx1, st1)

        def body(jj, _):
            i0 = 2 * jj
            for (i, xv, tv, sx, st) in ((i0, xv0, tv0, sx0, st0),
                                        (i0 + 1, xv1, tv1, sx1, st1)):
                process(i, xv, tv, sx, st)

                @pl.when(i + 2 < n_steps)
                def _(i=i, xv=xv, tv=tv, sx=sx, st=st):
                    issue(i + 2, xv, tv, sx, st)
            return 0

        lax.fori_loop(0, max(n_steps // 2, 1), body, 0)
        for r in range(B * 3):
            pltpu.sync_copy(outv.at[pl.ds(r * _L, _L)],
                            out_hbm.at[r, pl.ds(wid * _L, _L)])

    return sc_sums


def _tc_sums_body(x_ref, t_ref, o_ref, o2_ref):
    d = pl.program_id(1)
    xb = x_ref[0, 0]
    m = (t_ref[0] == 3).astype(jnp.float32)
    xr = xb.reshape(-1, 8, 128)
    mr = m.reshape(-1, 8, 128)

    @pl.when(d == 0)
    def _():
        o_ref[...] = jnp.zeros_like(o_ref)

    o_ref[0, 0] += xr.sum(0)
    o_ref[0, 1] += (xr * mr).sum(0)
    o_ref[0, 2] += mr.sum(0)

    @pl.when(d == pl.num_programs(1) - 1)
    def _():
        for k in range(3):
            o2_ref[0, 0, k] = jnp.sum(o_ref[0, k])


def kernel(inputs, targets):
    B, C, D, H, W = inputs.shape
    V = D * H * W
    v_sc = _DSC * H * W
    xf = inputs.reshape(-1)
    tf = targets.reshape(-1)
    sc_part = _make_sc_sums(B, C, V, v_sc)(xf, tf)
    d_off = _DSC // _DBLK
    _, tc_sums = pl.pallas_call(
        _tc_sums_body,
        grid=(B, (D - _DSC) // _DBLK),
        in_specs=[
            pl.BlockSpec((1, 1, _DBLK, H, W),
                         lambda b, d: (b, C - 1, d + d_off, 0, 0)),
            pl.BlockSpec((1, _DBLK, H, W), lambda b, d: (b, d + d_off, 0, 0)),
        ],
        out_specs=[
            pl.BlockSpec((1, 3, 8, 128), lambda b, d: (b, 0, 0, 0)),
            pl.BlockSpec(memory_space=pltpu.SMEM, block_shape=(1, 1, 3),
                         index_map=lambda b, d: (b, 0, 0)),
        ],
        out_shape=[
            jax.ShapeDtypeStruct((B, 3, 8, 128), jnp.float32),
            jax.ShapeDtypeStruct((B, 1, 3), jnp.float32),
        ],
    )(inputs, targets)
    sums = sc_part.sum(axis=1).reshape(B, 3) + tc_sums[:, 0]
    S, T, N = sums[:, 0], sums[:, 1], sums[:, 2]
    tversky = (T + _SMOOTH) / (T + _ALPHA * (N - T) + _BETA * (S - T) + _SMOOTH)
    return -tversky.mean()


# speedup vs baseline: 1.2442x; 1.0229x over previous
import functools

import jax
import jax.numpy as jnp
from jax import lax
from jax.experimental import pallas as pl
from jax.experimental.pallas import tpu as pltpu, tpu_sc as plsc

_ALPHA = 0.7
_BETA = 0.3
_SMOOTH = 1.0

_INFO = plsc.get_sparse_core_info()
_NC, _NS, _L = _INFO.num_cores, _INFO.num_subcores, _INFO.num_lanes
_NW = _NC * _NS
_DSC = 24
_DBLK = 8


def _make_sc_sums(B, C, V, v_sc):
    per_w = v_sc // _NW
    ch = per_w
    while ch > 8192:
        ch //= 2
    n_ch = per_w // ch
    n_steps = B * n_ch
    mesh = plsc.VectorSubcoreMesh(core_axis_name="c", subcore_axis_name="s")
    scratch = (
        [pltpu.VMEM((ch,), jnp.float32) for _ in range(2)]
        + [pltpu.VMEM((ch,), jnp.int32) for _ in range(2)]
        + [pltpu.VMEM((B * 3 * _L,), jnp.float32)]
        + [pltpu.SemaphoreType.DMA for _ in range(4)]
    )

    @functools.partial(
        pl.kernel,
        mesh=mesh,
        out_type=jax.ShapeDtypeStruct((B * 3, _NW * _L), jnp.float32),
        scratch_types=scratch,
    )
    def sc_sums(x_hbm, t_hbm, out_hbm, xv0, xv1, tv0, tv1, outv,
                sx0, sx1, st0, st1):
        wid = lax.axis_index("s") * _NC + lax.axis_index("c")
        base = wid * per_w
        zero = jnp.zeros((_L,), jnp.float32)
        for r in range(B * 3):
            outv[pl.ds(r * _L, _L)] = zero

        def issue(i, xv, tv, sx, st):
            b = i // n_ch
            off = base + (i - b * n_ch) * ch
            pltpu.async_copy(
                x_hbm.at[pl.ds((b * C + (C - 1)) * V + off, ch)], xv, sx)
            pltpu.async_copy(t_hbm.at[pl.ds(b * V + off, ch)], tv, st)

        def process(i, xv, tv, sx, st):
            pltpu.make_async_copy(x_hbm.at[pl.ds(0, ch)], xv, sx).wait()
            pltpu.make_async_copy(t_hbm.at[pl.ds(0, ch)], tv, st).wait()

            def inner(j, carry):
                s, t, n = carry
                xs = xv[pl.ds(j * _L, _L)]
                ts = tv[pl.ds(j * _L, _L)]
                m = ts == (C - 1)
                s = s + xs
                t = t + jnp.where(m, xs, 0.0)
                n = n + jnp.where(m, 1.0, 0.0)
                return s, t, n

            S, T, N = lax.fori_loop(0, ch // _L, inner, (zero, zero, zero),
                                    unroll=4)
            boff = (i // n_ch) * 3 * _L
            outv[pl.ds(boff, _L)] = outv[pl.ds(boff, _L)] + S
            outv[pl.ds(boff + _L, _L)] = outv[pl.ds(boff + _L, _L)] + T
            outv[pl.ds(boff + 2 * _L, _L)] = outv[pl.ds(boff + 2 * _L, _L)] + N

        issue(0, xv0, tv0, sx0, st0)
        if n_steps > 1:
            issue(1, xv1, tv1, sx1, st1)

        def body(jj, _):
            i0 = 2 * jj
            for (i, xv, tv, sx, st) in ((i0, xv0, tv0, sx0, st0),
                                        (i0 + 1, xv1, tv1, sx1, st1)):
                process(i, xv, tv, sx, st)

                @pl.when(i + 2 < n_steps)
                def _(i=i, xv=xv, tv=tv, sx=sx, st=st):
                    issue(i + 2, xv, tv, sx, st)
            return 0

        lax.fori_loop(0, max(n_steps // 2, 1), body, 0)
        for r in range(B * 3):
            pltpu.sync_copy(outv.at[pl.ds(r * _L, _L)],
                            out_hbm.at[r, pl.ds(wid * _L, _L)])

    return sc_sums


def _tc_sums_body(x_ref, t_ref, o_ref, o2_ref):
    d = pl.program_id(1)
    xb = x_ref[0, 0]
    m = (t_ref[0] == 3).astype(jnp.float32)
    xr = xb.reshape(-1, 8, 128)
    mr = m.reshape(-1, 8, 128)

    @pl.when(d == 0)
    def _():
        o_ref[...] = jnp.zeros_like(o_ref)

    o_ref[0, 0] += xr.sum(0)
    o_ref[0, 1] += (xr * mr).sum(0)
    o_ref[0, 2] += mr.sum(0)

    @pl.when(d == pl.num_programs(1) - 1)
    def _():
        for k in range(3):
            o2_ref[0, 0, k] = jnp.sum(o_ref[0, k])


def _make_combine_body(B):
    def _combine_body(sc_ref, tcs_ref, o_ref):
        acc = 0.0
        for b in range(B):
            S = jnp.sum(sc_ref[3 * b + 0]) + tcs_ref[b, 0, 0]
            T = jnp.sum(sc_ref[3 * b + 1]) + tcs_ref[b, 0, 1]
            N = jnp.sum(sc_ref[3 * b + 2]) + tcs_ref[b, 0, 2]
            acc += (T + _SMOOTH) / (
                T + _ALPHA * (N - T) + _BETA * (S - T) + _SMOOTH)
        o_ref[0, 0] = -acc / B

    return _combine_body


def kernel(inputs, targets):
    B, C, D, H, W = inputs.shape
    V = D * H * W
    v_sc = _DSC * H * W
    xf = inputs.reshape(-1)
    tf = targets.reshape(-1)
    sc_part = _make_sc_sums(B, C, V, v_sc)(xf, tf)
    d_off = _DSC // _DBLK
    _, tc_sums = pl.pallas_call(
        _tc_sums_body,
        grid=(B, (D - _DSC) // _DBLK),
        in_specs=[
            pl.BlockSpec((1, 1, _DBLK, H, W),
                         lambda b, d: (b, C - 1, d + d_off, 0, 0)),
            pl.BlockSpec((1, _DBLK, H, W), lambda b, d: (b, d + d_off, 0, 0)),
        ],
        out_specs=[
            pl.BlockSpec((1, 3, 8, 128), lambda b, d: (b, 0, 0, 0)),
            pl.BlockSpec(memory_space=pltpu.SMEM, block_shape=(1, 1, 3),
                         index_map=lambda b, d: (b, 0, 0)),
        ],
        out_shape=[
            jax.ShapeDtypeStruct((B, 3, 8, 128), jnp.float32),
            jax.ShapeDtypeStruct((B, 1, 3), jnp.float32),
        ],
    )(inputs, targets)
    loss = pl.pallas_call(
        _make_combine_body(B),
        in_specs=[
            pl.BlockSpec(memory_space=pltpu.VMEM),
            pl.BlockSpec(memory_space=pltpu.SMEM),
        ],
        out_specs=pl.BlockSpec(memory_space=pltpu.SMEM),
        out_shape=jax.ShapeDtypeStruct((1, 1), jnp.float32),
    )(sc_part, tc_sums)
    return loss[0, 0]
